# baseline (speedup 1.0000x reference)
import functools

import jax
import jax.numpy as jnp
from jax import lax
from jax.experimental import pallas as pl
from jax.experimental.pallas import tpu as pltpu
from jax.experimental.pallas import tpu_sc as plsc

VOCAB = 8192
D = 8192
K = 4
NBUF = 3
LOOK = 2


def _body(x_hbm, table_hbm, out_hbm, idx_v, *rest, nchunks, seq):
    bufs = rest[:NBUF]
    gsems = rest[NBUF:2 * NBUF]
    ssems = rest[2 * NBUF:3 * NBUF]
    W = NBUF - LOOK

    cid = lax.axis_index("c")
    sid = lax.axis_index("s")
    wid = sid * 2 + cid
    base = wid * nchunks
    wpb = seq // (nchunks * K)
    b_out = wid // wpb
    s_base = (wid % wpb) * (nchunks * K)

    pltpu.sync_copy(x_hbm.at[pl.ds(base, nchunks)], idx_v)

    def gather(j, b):
        return pltpu.make_async_copy(
            table_hbm.at[idx_v.at[j]], bufs[b], gsems[b])

    def scatter(j, b):
        return pltpu.make_async_copy(
            bufs[b], out_hbm.at[b_out, pl.ds(s_base + j * K, K)], ssems[b])

    for j in range(LOOK):
        gather(j, j % NBUF).start()

    def chunk_step(j, b, jg=None):
        jj = j if jg is None else jg
        gather(jj, b).wait()
        scatter(jj, b).start()
        scatter(jj - W, (b + LOOK) % NBUF).wait()
        gather(jj + LOOK, (b + LOOK) % NBUF).start()

    for j in range(NBUF):
        b = j % NBUF
        if j >= W:
            chunk_step(j, b)
        else:
            gather(j, b).wait()
            scatter(j, b).start()
            gather(j + LOOK, (b + LOOK) % NBUF).start()

    ngroups = nchunks // NBUF
    rem = nchunks - ngroups * NBUF

    def step(g, carry):
        j0 = g * NBUF
        for b in range(NBUF):
            chunk_step(b, b, jg=j0 + b)
        return carry

    nsteady = ngroups if rem >= LOOK else ngroups - 1
    lax.fori_loop(1, nsteady, step, 0)

    for j in range(nsteady * NBUF, nchunks):
        b = j % NBUF
        if j + LOOK < nchunks:
            chunk_step(j, b)
        else:
            gather(j, b).wait()
            scatter(j, b).start()

    for b in range(NBUF):
        scatter(0, b).wait()


def kernel(x, table):
    B, S = x.shape
    n = B * S
    info = plsc.get_sparse_core_info()
    nw = info.num_cores * info.num_subcores
    nchunks = n // (nw * K)

    xf = x.reshape(nw * nchunks, K).astype(jnp.int32)

    mesh = plsc.VectorSubcoreMesh(core_axis_name="c", subcore_axis_name="s")
    out = pl.kernel(
        functools.partial(_body, nchunks=nchunks, seq=S),
        out_type=jax.ShapeDtypeStruct((B, S, D), jnp.float32),
        mesh=mesh,
        scratch_types=(
            [pltpu.VMEM((nchunks, K), jnp.int32)]
            + [pltpu.VMEM((K, D), jnp.float32)] * NBUF
            + [pltpu.SemaphoreType.DMA] * (2 * NBUF)
        ),
    )(xf, table)
    return out
